# Initial kernel scaffold; baseline (speedup 1.0000x reference)
import functools

import jax
import jax.numpy as jnp
from jax import lax
from jax.experimental import pallas as pl
from jax.experimental.pallas import tpu as pltpu
from jax.experimental.pallas import tpu_sc as plsc

NC = 2
NS = 16
NW = NC * NS
LANE = 16
EDGE_BLK = 128


def _make_sc_agg(n_pad, nblk):
    rows_per_tile = n_pad // NS
    nz = rows_per_tile // EDGE_BLK
    mesh = plsc.VectorSubcoreMesh(
        core_axis_name="c", subcore_axis_name="s", num_cores=NC, num_subcores=NS)

    @functools.partial(
        pl.kernel, mesh=mesh,
        out_type=jax.ShapeDtypeStruct((NC, n_pad, LANE), jnp.float32),
        scratch_types=[
            pltpu.VMEM((nblk, EDGE_BLK), jnp.int32),
            pltpu.VMEM((nblk, EDGE_BLK), jnp.int32),
            pltpu.VMEM((EDGE_BLK, LANE), jnp.float32),
            pltpu.VMEM((EDGE_BLK, LANE), jnp.float32),
            pltpu.VMEM_SHARED((n_pad, LANE), jnp.float32),
            pltpu.SemaphoreType.DMA,
        ],
    )
    def kern(table_hbm, src_hbm, dst_hbm, out_hbm, sidx, didx, rows, zbuf, acc, sem):
        c = lax.axis_index("c")
        s = lax.axis_index("s")
        wid = s * NC + c
        base = s * rows_per_tile

        def fill(i, _):
            zbuf[i] = jnp.zeros((LANE,), jnp.float32)
            return 0
        lax.fori_loop(0, EDGE_BLK, fill, 0)

        def zloop(k, _):
            pltpu.sync_copy(zbuf, acc.at[pl.ds(base + k * EDGE_BLK, EDGE_BLK)])
            return 0
        lax.fori_loop(0, nz, zloop, 0)

        pltpu.sync_copy(src_hbm.at[wid], sidx)
        pltpu.sync_copy(dst_hbm.at[wid], didx)
        plsc.subcore_barrier()

        def body(j, _):
            pltpu.async_copy(table_hbm.at[sidx.at[j]], rows, sem).wait()
            pltpu.sync_copy(rows, acc.at[didx.at[j]], add=True)
            return 0
        lax.fori_loop(0, nblk, body, 0)

        plsc.subcore_barrier()
        pltpu.sync_copy(acc.at[pl.ds(base, rows_per_tile)],
                        out_hbm.at[c, pl.ds(base, rows_per_tile)])

    return kern


def _make_sc_deg(n_pad, nblk):
    rows_per_tile = n_pad // NS
    nz = rows_per_tile // EDGE_BLK
    mesh = plsc.VectorSubcoreMesh(
        core_axis_name="c", subcore_axis_name="s", num_cores=NC, num_subcores=NS)

    @functools.partial(
        pl.kernel, mesh=mesh,
        out_type=jax.ShapeDtypeStruct((NC, n_pad, LANE), jnp.float32),
        scratch_types=[
            pltpu.VMEM((nblk, EDGE_BLK), jnp.int32),
            pltpu.VMEM((EDGE_BLK, LANE), jnp.float32),
            pltpu.VMEM((EDGE_BLK, LANE), jnp.float32),
            pltpu.VMEM_SHARED((n_pad, LANE), jnp.float32),
        ],
    )
    def kern(dst_hbm, out_hbm, didx, ones, zbuf, acc):
        c = lax.axis_index("c")
        s = lax.axis_index("s")
        wid = s * NC + c
        base = s * rows_per_tile

        def fill(i, _):
            zbuf[i] = jnp.zeros((LANE,), jnp.float32)
            ones[i] = jnp.ones((LANE,), jnp.float32)
            return 0
        lax.fori_loop(0, EDGE_BLK, fill, 0)

        def zloop(k, _):
            pltpu.sync_copy(zbuf, acc.at[pl.ds(base + k * EDGE_BLK, EDGE_BLK)])
            return 0
        lax.fori_loop(0, nz, zloop, 0)

        pltpu.sync_copy(dst_hbm.at[wid], didx)
        plsc.subcore_barrier()

        def body(j, _):
            pltpu.sync_copy(ones, acc.at[didx.at[j]], add=True)
            return 0
        lax.fori_loop(0, nblk, body, 0)

        plsc.subcore_barrier()
        pltpu.sync_copy(acc.at[pl.ds(base, rows_per_tile)],
                        out_hbm.at[c, pl.ds(base, rows_per_tile)])

    return kern


def _tc_prescale(x_p, W1, degp):
    n_pad = x_p.shape[0]

    def body(x_ref, w_ref, d_ref, hs_ref, dinv_ref):
        deg = d_ref[0] + d_ref[1] + 1.0
        dinv = lax.rsqrt(deg)
        h = lax.dot_general(x_ref[...], w_ref[...], (((1,), (1,)), ((), ())),
                            preferred_element_type=jnp.float32)
        hs_ref[...] = h * dinv
        dinv_ref[...] = dinv

    return pl.pallas_call(
        body,
        out_shape=(jax.ShapeDtypeStruct((n_pad, LANE), jnp.float32),
                   jax.ShapeDtypeStruct((n_pad, LANE), jnp.float32)),
    )(x_p, W1, degp)


def _tc_mid(p1, hs1, dinv, b1, W2, n_valid):
    n_pad = hs1.shape[0]

    def body(p_ref, hs_ref, dinv_ref, b_ref, w_ref, o_ref):
        out1 = jnp.maximum(
            dinv_ref[...] * (p_ref[0] + p_ref[1] + hs_ref[...]) + b_ref[...], 0.0)
        h2 = lax.dot_general(out1, w_ref[...], (((1,), (1,)), ((), ())),
                             preferred_element_type=jnp.float32)
        rid = lax.broadcasted_iota(jnp.int32, (n_pad, LANE), 0)
        o_ref[...] = jnp.where(rid < n_valid, h2 * dinv_ref[...], 0.0)

    return pl.pallas_call(
        body,
        out_shape=jax.ShapeDtypeStruct((n_pad, LANE), jnp.float32),
    )(p1, hs1, dinv, b1, W2)


def _tc_final(p2, hs2, dinv, b2):
    n_pad = hs2.shape[0]

    def body(p_ref, hs_ref, dinv_ref, b_ref, o_ref):
        o_ref[...] = dinv_ref[...] * (p_ref[0] + p_ref[1] + hs_ref[...]) + b_ref[...]

    return pl.pallas_call(
        body,
        out_shape=jax.ShapeDtypeStruct((n_pad, LANE), jnp.float32),
    )(p2, hs2, dinv, b2)


def kernel(x, edge_index, W1, b1, W2, b2):
    n, _ = x.shape
    e = edge_index.shape[1]

    n_pad = ((n + NS * EDGE_BLK - 1) // (NS * EDGE_BLK)) * (NS * EDGE_BLK)
    ew = ((e + NW - 1) // NW + EDGE_BLK - 1) // EDGE_BLK * EDGE_BLK
    nblk = ew // EDGE_BLK
    e_pad = ew * NW
    sentinel = n_pad - 1

    src = edge_index[0].astype(jnp.int32)
    dst = edge_index[1].astype(jnp.int32)
    pad = jnp.full((e_pad - e,), sentinel, jnp.int32)
    src_p = jnp.concatenate([src, pad]).reshape(NW, nblk, EDGE_BLK)
    dst_p = jnp.concatenate([dst, pad]).reshape(NW, nblk, EDGE_BLK)
    x_p = jnp.pad(x, ((0, n_pad - n), (0, 0)))
    b1r = b1.reshape(1, LANE)
    b2r = b2.reshape(1, LANE)

    sc_deg = _make_sc_deg(n_pad, nblk)
    sc_agg = _make_sc_agg(n_pad, nblk)

    degp = sc_deg(dst_p)
    hs1, dinv = _tc_prescale(x_p, W1, degp)
    p1 = sc_agg(hs1, src_p, dst_p)
    hs2 = _tc_mid(p1, hs1, dinv, b1r, W2, n)
    p2 = sc_agg(hs2, src_p, dst_p)
    out = _tc_final(p2, hs2, dinv, b2r)
    return out[:n]


# speedup vs baseline: 51.8551x; 51.8551x over previous
import functools

import jax
import jax.numpy as jnp
from jax import lax
from jax.experimental import pallas as pl
from jax.experimental.pallas import tpu as pltpu
from jax.experimental.pallas import tpu_sc as plsc

NC = 2
NS = 16
NW = NC * NS
LANE = 16
EDGE_BLK = 128


def _make_sc_agg(n_pad, nblk):
    rows_per_tile = n_pad // NS
    nz = rows_per_tile // EDGE_BLK
    mesh = plsc.VectorSubcoreMesh(
        core_axis_name="c", subcore_axis_name="s", num_cores=NC, num_subcores=NS)

    @functools.partial(
        pl.kernel, mesh=mesh,
        out_type=jax.ShapeDtypeStruct((NC, n_pad, LANE), jnp.float32),
        scratch_types=[
            pltpu.VMEM((nblk, EDGE_BLK), jnp.int32),
            pltpu.VMEM((nblk, EDGE_BLK), jnp.int32),
            pltpu.VMEM((EDGE_BLK, LANE), jnp.float32),
            pltpu.VMEM((EDGE_BLK, LANE), jnp.float32),
            pltpu.VMEM_SHARED((n_pad, LANE), jnp.float32),
            pltpu.VMEM_SHARED((n_pad, LANE), jnp.float32),
            pltpu.SemaphoreType.DMA,
        ],
        compiler_params=pltpu.CompilerParams(use_tc_tiling_on_sc=False),
    )
    def kern(table_hbm, src_hbm, dst_hbm, out_hbm, sidx, didx, rows, zbuf, tbl, acc, sem):
        c = lax.axis_index("c")
        s = lax.axis_index("s")
        wid = s * NC + c
        base = s * rows_per_tile

        def fill(i, _):
            zbuf[i] = jnp.zeros((LANE,), jnp.float32)
            return 0
        lax.fori_loop(0, EDGE_BLK, fill, 0)

        def zloop(k, _):
            pltpu.sync_copy(zbuf, acc.at[pl.ds(base + k * EDGE_BLK, EDGE_BLK)])
            return 0
        lax.fori_loop(0, nz, zloop, 0)

        pltpu.sync_copy(table_hbm.at[pl.ds(base, rows_per_tile)],
                        tbl.at[pl.ds(base, rows_per_tile)])
        pltpu.sync_copy(src_hbm.at[wid], sidx)
        pltpu.sync_copy(dst_hbm.at[wid], didx)
        plsc.subcore_barrier()

        def body(j, _):
            pltpu.async_copy(tbl.at[sidx.at[j]], rows, sem).wait()
            pltpu.sync_copy(rows, acc.at[didx.at[j]], add=True)
            return 0
        lax.fori_loop(0, nblk, body, 0)

        plsc.subcore_barrier()
        pltpu.sync_copy(acc.at[pl.ds(base, rows_per_tile)],
                        out_hbm.at[c, pl.ds(base, rows_per_tile)])

    return kern


def _make_sc_deg(n_pad, nblk):
    rows_per_tile = n_pad // NS
    nz = rows_per_tile // EDGE_BLK
    mesh = plsc.VectorSubcoreMesh(
        core_axis_name="c", subcore_axis_name="s", num_cores=NC, num_subcores=NS)

    @functools.partial(
        pl.kernel, mesh=mesh,
        out_type=jax.ShapeDtypeStruct((NC, n_pad, LANE), jnp.float32),
        scratch_types=[
            pltpu.VMEM((nblk, EDGE_BLK), jnp.int32),
            pltpu.VMEM((EDGE_BLK, LANE), jnp.float32),
            pltpu.VMEM((EDGE_BLK, LANE), jnp.float32),
            pltpu.VMEM_SHARED((n_pad, LANE), jnp.float32),
        ],
        compiler_params=pltpu.CompilerParams(use_tc_tiling_on_sc=False),
    )
    def kern(dst_hbm, out_hbm, didx, ones, zbuf, acc):
        c = lax.axis_index("c")
        s = lax.axis_index("s")
        wid = s * NC + c
        base = s * rows_per_tile

        def fill(i, _):
            zbuf[i] = jnp.zeros((LANE,), jnp.float32)
            ones[i] = jnp.ones((LANE,), jnp.float32)
            return 0
        lax.fori_loop(0, EDGE_BLK, fill, 0)

        def zloop(k, _):
            pltpu.sync_copy(zbuf, acc.at[pl.ds(base + k * EDGE_BLK, EDGE_BLK)])
            return 0
        lax.fori_loop(0, nz, zloop, 0)

        pltpu.sync_copy(dst_hbm.at[wid], didx)
        plsc.subcore_barrier()

        def body(j, _):
            pltpu.sync_copy(ones, acc.at[didx.at[j]], add=True)
            return 0
        lax.fori_loop(0, nblk, body, 0)

        plsc.subcore_barrier()
        pltpu.sync_copy(acc.at[pl.ds(base, rows_per_tile)],
                        out_hbm.at[c, pl.ds(base, rows_per_tile)])

    return kern


def _tc_prescale(x_p, W1, degp):
    n_pad = x_p.shape[0]

    def body(x_ref, w_ref, d_ref, hs_ref, dinv_ref):
        deg = d_ref[0] + d_ref[1] + 1.0
        dinv = lax.rsqrt(deg)
        h = lax.dot_general(x_ref[...], w_ref[...], (((1,), (1,)), ((), ())),
                            preferred_element_type=jnp.float32)
        hs_ref[...] = h * dinv
        dinv_ref[...] = dinv

    return pl.pallas_call(
        body,
        out_shape=(jax.ShapeDtypeStruct((n_pad, LANE), jnp.float32),
                   jax.ShapeDtypeStruct((n_pad, LANE), jnp.float32)),
    )(x_p, W1, degp)


def _tc_mid(p1, hs1, dinv, b1, W2, n_valid):
    n_pad = hs1.shape[0]

    def body(p_ref, hs_ref, dinv_ref, b_ref, w_ref, o_ref):
        out1 = jnp.maximum(
            dinv_ref[...] * (p_ref[0] + p_ref[1] + hs_ref[...]) + b_ref[...], 0.0)
        h2 = lax.dot_general(out1, w_ref[...], (((1,), (1,)), ((), ())),
                             preferred_element_type=jnp.float32)
        rid = lax.broadcasted_iota(jnp.int32, (n_pad, LANE), 0)
        o_ref[...] = jnp.where(rid < n_valid, h2 * dinv_ref[...], 0.0)

    return pl.pallas_call(
        body,
        out_shape=jax.ShapeDtypeStruct((n_pad, LANE), jnp.float32),
    )(p1, hs1, dinv, b1, W2)


def _tc_final(p2, hs2, dinv, b2):
    n_pad = hs2.shape[0]

    def body(p_ref, hs_ref, dinv_ref, b_ref, o_ref):
        o_ref[...] = dinv_ref[...] * (p_ref[0] + p_ref[1] + hs_ref[...]) + b_ref[...]

    return pl.pallas_call(
        body,
        out_shape=jax.ShapeDtypeStruct((n_pad, LANE), jnp.float32),
    )(p2, hs2, dinv, b2)


def kernel(x, edge_index, W1, b1, W2, b2):
    n, _ = x.shape
    e = edge_index.shape[1]

    n_pad = ((n + NS * EDGE_BLK - 1) // (NS * EDGE_BLK)) * (NS * EDGE_BLK)
    ew = ((e + NW - 1) // NW + 8 * EDGE_BLK - 1) // (8 * EDGE_BLK) * (8 * EDGE_BLK)
    nblk = ew // EDGE_BLK
    e_pad = ew * NW
    sentinel = n_pad - 1

    src = edge_index[0].astype(jnp.int32)
    dst = edge_index[1].astype(jnp.int32)
    pad = jnp.full((e_pad - e,), sentinel, jnp.int32)
    src_p = jnp.concatenate([src, pad]).reshape(NW, nblk, EDGE_BLK)
    dst_p = jnp.concatenate([dst, pad]).reshape(NW, nblk, EDGE_BLK)
    x_p = jnp.pad(x, ((0, n_pad - n), (0, 0)))
    b1r = b1.reshape(1, LANE)
    b2r = b2.reshape(1, LANE)

    sc_deg = _make_sc_deg(n_pad, nblk)
    sc_agg = _make_sc_agg(n_pad, nblk)

    degp = sc_deg(dst_p)
    hs1, dinv = _tc_prescale(x_p, W1, degp)
    p1 = sc_agg(hs1, src_p, dst_p)
    hs2 = _tc_mid(p1, hs1, dinv, b1r, W2, n)
    p2 = sc_agg(hs2, src_p, dst_p)
    out = _tc_final(p2, hs2, dinv, b2r)
    return out[:n]


# speedup vs baseline: 59.3490x; 1.1445x over previous
import functools

import jax
import jax.numpy as jnp
from jax import lax
from jax.experimental import pallas as pl
from jax.experimental.pallas import tpu as pltpu
from jax.experimental.pallas import tpu_sc as plsc

NC = 2
NS = 16
NW = NC * NS
LANE = 16
EDGE_BLK = 125
KBUF = 8


def _make_sc_agg(n_pad, nblk):
    rows_per_tile = n_pad // NS
    nz = rows_per_tile // EDGE_BLK
    mesh = plsc.VectorSubcoreMesh(
        core_axis_name="c", subcore_axis_name="s", num_cores=NC, num_subcores=NS)

    @functools.partial(
        pl.kernel, mesh=mesh,
        out_type=jax.ShapeDtypeStruct((NC, n_pad, LANE), jnp.float32),
        scratch_types=[
            pltpu.VMEM((nblk, EDGE_BLK), jnp.int32),
            pltpu.VMEM((nblk, EDGE_BLK), jnp.int32),
            pltpu.VMEM((KBUF, EDGE_BLK, LANE), jnp.float32),
            pltpu.VMEM((EDGE_BLK, LANE), jnp.float32),
            pltpu.VMEM_SHARED((n_pad, LANE), jnp.float32),
            pltpu.VMEM_SHARED((n_pad, LANE), jnp.float32),
            pltpu.SemaphoreType.DMA,
            pltpu.SemaphoreType.DMA,
        ],
        compiler_params=pltpu.CompilerParams(use_tc_tiling_on_sc=False),
    )
    def kern(table_hbm, src_hbm, dst_hbm, out_hbm,
             sidx, didx, rows, zbuf, tbl, acc, sem, stage_sem):
        c = lax.axis_index("c")
        s = lax.axis_index("s")
        wid = s * NC + c
        base = s * rows_per_tile

        stage = [
            pltpu.async_copy(table_hbm.at[pl.ds(base, rows_per_tile)],
                             tbl.at[pl.ds(base, rows_per_tile)], stage_sem),
            pltpu.async_copy(src_hbm.at[wid], sidx, stage_sem),
            pltpu.async_copy(dst_hbm.at[wid], didx, stage_sem),
        ]

        def fill(i, _):
            zbuf[i] = jnp.zeros((LANE,), jnp.float32)
            return 0
        lax.fori_loop(0, EDGE_BLK, fill, 0)

        def zloop(k, _):
            pltpu.sync_copy(zbuf, acc.at[pl.ds(base + k * EDGE_BLK, EDGE_BLK)])
            return 0
        lax.fori_loop(0, nz, zloop, 0)

        for d in stage:
            d.wait()
        plsc.subcore_barrier()

        def group(g, _):
            descs = []
            for b in range(KBUF):
                j = g * KBUF + b
                descs.append(
                    pltpu.async_copy(tbl.at[sidx.at[j]], rows.at[b], sem))
            for b in range(KBUF):
                descs[b].wait()
            for b in range(KBUF):
                j = g * KBUF + b
                pltpu.sync_copy(rows.at[b], acc.at[didx.at[j]], add=True)
            return 0
        lax.fori_loop(0, nblk // KBUF, group, 0)

        plsc.subcore_barrier()
        pltpu.sync_copy(acc.at[pl.ds(base, rows_per_tile)],
                        out_hbm.at[c, pl.ds(base, rows_per_tile)])

    return kern


def _make_sc_deg(n_pad, nblk):
    rows_per_tile = n_pad // NS
    nz = rows_per_tile // EDGE_BLK
    mesh = plsc.VectorSubcoreMesh(
        core_axis_name="c", subcore_axis_name="s", num_cores=NC, num_subcores=NS)

    @functools.partial(
        pl.kernel, mesh=mesh,
        out_type=jax.ShapeDtypeStruct((NC, n_pad, LANE), jnp.float32),
        scratch_types=[
            pltpu.VMEM((nblk, EDGE_BLK), jnp.int32),
            pltpu.VMEM((EDGE_BLK, LANE), jnp.float32),
            pltpu.VMEM((EDGE_BLK, LANE), jnp.float32),
            pltpu.VMEM_SHARED((n_pad, LANE), jnp.float32),
            pltpu.SemaphoreType.DMA,
        ],
        compiler_params=pltpu.CompilerParams(use_tc_tiling_on_sc=False),
    )
    def kern(dst_hbm, out_hbm, didx, ones, zbuf, acc, sem):
        c = lax.axis_index("c")
        s = lax.axis_index("s")
        wid = s * NC + c
        base = s * rows_per_tile

        stage = pltpu.async_copy(dst_hbm.at[wid], didx, sem)

        def fill(i, _):
            zbuf[i] = jnp.zeros((LANE,), jnp.float32)
            ones[i] = jnp.ones((LANE,), jnp.float32)
            return 0
        lax.fori_loop(0, EDGE_BLK, fill, 0)

        def zloop(k, _):
            pltpu.sync_copy(zbuf, acc.at[pl.ds(base + k * EDGE_BLK, EDGE_BLK)])
            return 0
        lax.fori_loop(0, nz, zloop, 0)

        stage.wait()
        plsc.subcore_barrier()

        def group(g, _):
            descs = []
            for b in range(KBUF):
                j = g * KBUF + b
                descs.append(
                    pltpu.async_copy(ones, acc.at[didx.at[j]], sem, add=True))
            for b in range(KBUF):
                descs[b].wait()
            return 0
        lax.fori_loop(0, nblk // KBUF, group, 0)

        plsc.subcore_barrier()
        pltpu.sync_copy(acc.at[pl.ds(base, rows_per_tile)],
                        out_hbm.at[c, pl.ds(base, rows_per_tile)])

    return kern


def _tc_prescale(x, W1, degp):
    n = x.shape[0]

    def body(x_ref, w_ref, d_ref, hs_ref, dinv_ref):
        deg = d_ref[0] + d_ref[1] + 1.0
        dinv = lax.rsqrt(deg)
        h = lax.dot_general(x_ref[...], w_ref[...], (((1,), (1,)), ((), ())),
                            preferred_element_type=jnp.float32)
        hs_ref[...] = h * dinv
        dinv_ref[...] = dinv

    return pl.pallas_call(
        body,
        out_shape=(jax.ShapeDtypeStruct((n, LANE), jnp.float32),
                   jax.ShapeDtypeStruct((n, LANE), jnp.float32)),
    )(x, W1, degp)


def _tc_mid(p1, hs1, dinv, b1, W2):
    n = hs1.shape[0]

    def body(p_ref, hs_ref, dinv_ref, b_ref, w_ref, o_ref):
        out1 = jnp.maximum(
            dinv_ref[...] * (p_ref[0] + p_ref[1] + hs_ref[...]) + b_ref[...], 0.0)
        h2 = lax.dot_general(out1, w_ref[...], (((1,), (1,)), ((), ())),
                             preferred_element_type=jnp.float32)
        o_ref[...] = h2 * dinv_ref[...]

    return pl.pallas_call(
        body,
        out_shape=jax.ShapeDtypeStruct((n, LANE), jnp.float32),
    )(p1, hs1, dinv, b1, W2)


def _tc_final(p2, hs2, dinv, b2):
    n = hs2.shape[0]

    def body(p_ref, hs_ref, dinv_ref, b_ref, o_ref):
        o_ref[...] = dinv_ref[...] * (p_ref[0] + p_ref[1] + hs_ref[...]) + b_ref[...]

    return pl.pallas_call(
        body,
        out_shape=jax.ShapeDtypeStruct((n, LANE), jnp.float32),
    )(p2, hs2, dinv, b2)


def kernel(x, edge_index, W1, b1, W2, b2):
    n, _ = x.shape
    e = edge_index.shape[1]

    assert e % (NW * EDGE_BLK) == 0 and n % NS == 0
    ew = e // NW
    nblk = ew // EDGE_BLK
    assert nblk % KBUF == 0 and (n // NS) % EDGE_BLK == 0

    src_p = edge_index[0].astype(jnp.int32).reshape(NW, nblk, EDGE_BLK)
    dst_p = edge_index[1].astype(jnp.int32).reshape(NW, nblk, EDGE_BLK)
    b1r = b1.reshape(1, LANE)
    b2r = b2.reshape(1, LANE)

    sc_deg = _make_sc_deg(n, nblk)
    sc_agg = _make_sc_agg(n, nblk)

    degp = sc_deg(dst_p)
    hs1, dinv = _tc_prescale(x, W1, degp)
    p1 = sc_agg(hs1, src_p, dst_p)
    hs2 = _tc_mid(p1, hs1, dinv, b1r, W2)
    p2 = sc_agg(hs2, src_p, dst_p)
    return _tc_final(p2, hs2, dinv, b2r)


# speedup vs baseline: 64.9396x; 1.0942x over previous
import functools

import jax
import jax.numpy as jnp
from jax import lax
from jax.experimental import pallas as pl
from jax.experimental.pallas import tpu as pltpu
from jax.experimental.pallas import tpu_sc as plsc

NC = 2
NS = 16
NW = NC * NS
LANE = 16
EDGE_BLK = 125
KBUF = 8


def _make_sc_agg(n_pad, nblk):
    rows_per_tile = n_pad // NS
    nz = rows_per_tile // EDGE_BLK
    mesh = plsc.VectorSubcoreMesh(
        core_axis_name="c", subcore_axis_name="s", num_cores=NC, num_subcores=NS)

    @functools.partial(
        pl.kernel, mesh=mesh,
        out_type=jax.ShapeDtypeStruct((NC, n_pad, LANE), jnp.float32),
        scratch_types=[
            pltpu.VMEM((nblk, EDGE_BLK), jnp.int32),
            pltpu.VMEM((nblk, EDGE_BLK), jnp.int32),
            pltpu.VMEM((2 * KBUF, EDGE_BLK, LANE), jnp.float32),
            pltpu.VMEM((EDGE_BLK, LANE), jnp.float32),
            pltpu.VMEM_SHARED((n_pad, LANE), jnp.float32),
            pltpu.VMEM_SHARED((n_pad, LANE), jnp.float32),
            pltpu.SemaphoreType.DMA,
            pltpu.SemaphoreType.DMA,
            pltpu.SemaphoreType.DMA,
        ],
        compiler_params=pltpu.CompilerParams(use_tc_tiling_on_sc=False),
    )
    def kern(table_hbm, src_hbm, dst_hbm, out_hbm,
             sidx, didx, rows, zbuf, tbl, acc, sem, ssem, stage_sem):
        c = lax.axis_index("c")
        s = lax.axis_index("s")
        wid = s * NC + c
        base = s * rows_per_tile

        stage = [
            pltpu.async_copy(table_hbm.at[pl.ds(base, rows_per_tile)],
                             tbl.at[pl.ds(base, rows_per_tile)], stage_sem),
            pltpu.async_copy(src_hbm.at[wid], sidx, stage_sem),
            pltpu.async_copy(dst_hbm.at[wid], didx, stage_sem),
        ]

        def fill(i, _):
            zbuf[i] = jnp.zeros((LANE,), jnp.float32)
            return 0
        lax.fori_loop(0, EDGE_BLK, fill, 0)

        def zloop(k, _):
            pltpu.sync_copy(zbuf, acc.at[pl.ds(base + k * EDGE_BLK, EDGE_BLK)])
            return 0
        lax.fori_loop(0, nz, zloop, 0)

        for d in stage:
            d.wait()
        plsc.subcore_barrier()

        ngroups = nblk // KBUF

        def group(g, _):
            half = (g % 2) * KBUF
            @pl.when(g >= 2)
            def _():
                for b in range(KBUF):
                    pltpu.make_async_copy(
                        rows.at[half + b], acc.at[didx.at[0]], ssem).wait()
            descs = []
            for b in range(KBUF):
                j = g * KBUF + b
                descs.append(
                    pltpu.async_copy(tbl.at[sidx.at[j]], rows.at[half + b], sem))
            for b in range(KBUF):
                descs[b].wait()
            for b in range(KBUF):
                j = g * KBUF + b
                pltpu.async_copy(rows.at[half + b], acc.at[didx.at[j]], ssem,
                                 add=True)
            return 0
        lax.fori_loop(0, ngroups, group, 0)

        for g in (ngroups - 2, ngroups - 1):
            half = (g % 2) * KBUF
            for b in range(KBUF):
                pltpu.make_async_copy(
                    rows.at[half + b], acc.at[didx.at[0]], ssem).wait()

        plsc.subcore_barrier()
        pltpu.sync_copy(acc.at[pl.ds(base, rows_per_tile)],
                        out_hbm.at[c, pl.ds(base, rows_per_tile)])

    return kern


def _make_sc_deg(n_pad, nblk):
    rows_per_tile = n_pad // NS
    nz = rows_per_tile // EDGE_BLK
    mesh = plsc.VectorSubcoreMesh(
        core_axis_name="c", subcore_axis_name="s", num_cores=NC, num_subcores=NS)

    @functools.partial(
        pl.kernel, mesh=mesh,
        out_type=jax.ShapeDtypeStruct((NC, n_pad, LANE), jnp.float32),
        scratch_types=[
            pltpu.VMEM((nblk, EDGE_BLK), jnp.int32),
            pltpu.VMEM((EDGE_BLK, LANE), jnp.float32),
            pltpu.VMEM((EDGE_BLK, LANE), jnp.float32),
            pltpu.VMEM_SHARED((n_pad, LANE), jnp.float32),
            pltpu.SemaphoreType.DMA,
        ],
        compiler_params=pltpu.CompilerParams(use_tc_tiling_on_sc=False),
    )
    def kern(dst_hbm, out_hbm, didx, ones, zbuf, acc, sem):
        c = lax.axis_index("c")
        s = lax.axis_index("s")
        wid = s * NC + c
        base = s * rows_per_tile

        stage = pltpu.async_copy(dst_hbm.at[wid], didx, sem)

        def fill(i, _):
            zbuf[i] = jnp.zeros((LANE,), jnp.float32)
            ones[i] = jnp.ones((LANE,), jnp.float32)
            return 0
        lax.fori_loop(0, EDGE_BLK, fill, 0)

        def zloop(k, _):
            pltpu.sync_copy(zbuf, acc.at[pl.ds(base + k * EDGE_BLK, EDGE_BLK)])
            return 0
        lax.fori_loop(0, nz, zloop, 0)

        stage.wait()
        plsc.subcore_barrier()

        def group(g, _):
            descs = []
            for b in range(KBUF):
                j = g * KBUF + b
                descs.append(
                    pltpu.async_copy(ones, acc.at[didx.at[j]], sem, add=True))
            for b in range(KBUF):
                descs[b].wait()
            return 0
        lax.fori_loop(0, nblk // KBUF, group, 0)

        plsc.subcore_barrier()
        pltpu.sync_copy(acc.at[pl.ds(base, rows_per_tile)],
                        out_hbm.at[c, pl.ds(base, rows_per_tile)])

    return kern


def _tc_prescale(x, W1, degp):
    n = x.shape[0]

    def body(x_ref, w_ref, d_ref, hs_ref, dinv_ref):
        deg = d_ref[0] + d_ref[1] + 1.0
        dinv = lax.rsqrt(deg)
        h = lax.dot_general(x_ref[...], w_ref[...], (((1,), (1,)), ((), ())),
                            preferred_element_type=jnp.float32)
        hs_ref[...] = h * dinv
        dinv_ref[...] = dinv

    return pl.pallas_call(
        body,
        out_shape=(jax.ShapeDtypeStruct((n, LANE), jnp.float32),
                   jax.ShapeDtypeStruct((n, LANE), jnp.float32)),
    )(x, W1, degp)


def _tc_mid(p1, hs1, dinv, b1, W2):
    n = hs1.shape[0]

    def body(p_ref, hs_ref, dinv_ref, b_ref, w_ref, o_ref):
        out1 = jnp.maximum(
            dinv_ref[...] * (p_ref[0] + p_ref[1] + hs_ref[...]) + b_ref[...], 0.0)
        h2 = lax.dot_general(out1, w_ref[...], (((1,), (1,)), ((), ())),
                             preferred_element_type=jnp.float32)
        o_ref[...] = h2 * dinv_ref[...]

    return pl.pallas_call(
        body,
        out_shape=jax.ShapeDtypeStruct((n, LANE), jnp.float32),
    )(p1, hs1, dinv, b1, W2)


def _tc_final(p2, hs2, dinv, b2):
    n = hs2.shape[0]

    def body(p_ref, hs_ref, dinv_ref, b_ref, o_ref):
        o_ref[...] = dinv_ref[...] * (p_ref[0] + p_ref[1] + hs_ref[...]) + b_ref[...]

    return pl.pallas_call(
        body,
        out_shape=jax.ShapeDtypeStruct((n, LANE), jnp.float32),
    )(p2, hs2, dinv, b2)


def kernel(x, edge_index, W1, b1, W2, b2):
    n, _ = x.shape
    e = edge_index.shape[1]

    assert e % (NW * EDGE_BLK) == 0 and n % NS == 0
    ew = e // NW
    nblk = ew // EDGE_BLK
    assert nblk % KBUF == 0 and (n // NS) % EDGE_BLK == 0

    src_p = edge_index[0].astype(jnp.int32).reshape(NW, nblk, EDGE_BLK)
    dst_p = edge_index[1].astype(jnp.int32).reshape(NW, nblk, EDGE_BLK)
    b1r = b1.reshape(1, LANE)
    b2r = b2.reshape(1, LANE)

    sc_deg = _make_sc_deg(n, nblk)
    sc_agg = _make_sc_agg(n, nblk)

    degp = sc_deg(dst_p)
    hs1, dinv = _tc_prescale(x, W1, degp)
    p1 = sc_agg(hs1, src_p, dst_p)
    hs2 = _tc_mid(p1, hs1, dinv, b1r, W2)
    p2 = sc_agg(hs2, src_p, dst_p)
    return _tc_final(p2, hs2, dinv, b2r)
